# baseline (speedup 1.0000x reference)
import functools

import jax
import jax.numpy as jnp
from jax import lax
from jax.experimental import pallas as pl
from jax.experimental.pallas import tpu as pltpu
from jax.experimental.pallas import tpu_sc as plsc

B = 1024
C = 100000
K = 5
ALPHA = 1.0
TOPS = K + 1
REQ = TOPS + 1

STR = 8
NSTR = C // STR
CBLK = 2048
NCB = -(-C // CBLK)
SPB = CBLK // STR

NW = 32
SPW = B // NW
HALFS = SPW // 2

FB = 128

NEG = float("-inf")


def _stripemax_body(x_ref, v_ref, i_ref):
    j = pl.program_id(0)
    xb = x_ref[...]
    x3 = xb.reshape(SPB, STR, B)

    def _top6_of(sm):
        iot = lax.broadcasted_iota(jnp.int32, (SPB, B), 0)
        big = jnp.int32(2**30)
        for t in range(TOPS):
            m = jnp.max(sm, axis=0, keepdims=True)
            idx = jnp.min(jnp.where(sm == m, iot, big), axis=0, keepdims=True)
            v_ref[t:t + 1, :] = m
            i_ref[t:t + 1, :] = idx + j * SPB
            sm = jnp.where(iot == idx, NEG, sm)
        v_ref[TOPS:8, :] = jnp.full((8 - TOPS, B), NEG, jnp.float32)
        i_ref[TOPS:8, :] = jnp.zeros((8 - TOPS, B), jnp.int32)

    @pl.when(j < NCB - 1)
    def _full():
        _top6_of(jnp.max(x3, axis=1))

    @pl.when(j == NCB - 1)
    def _tail():
        r = (lax.broadcasted_iota(jnp.int32, (SPB, STR, B), 0) * STR
             + lax.broadcasted_iota(jnp.int32, (SPB, STR, B), 1))
        xm = jnp.where(j * CBLK + r < C, x3, NEG)
        _top6_of(jnp.max(xm, axis=1))


def _stage_stripemax(xt):
    return pl.pallas_call(
        _stripemax_body,
        grid=(NCB,),
        in_specs=[pl.BlockSpec((CBLK, B), lambda j: (j, 0))],
        out_specs=[
            pl.BlockSpec((8, B), lambda j: (j, 0)),
            pl.BlockSpec((8, B), lambda j: (j, 0)),
        ],
        out_shape=[
            jax.ShapeDtypeStruct((NCB * 8, B), jnp.float32),
            jax.ShapeDtypeStruct((NCB * 8, B), jnp.int32),
        ],
    )(xt)


def _merge_body(v_ref, i_ref, o_ref):
    vals = v_ref[...]
    ids = i_ref[...]
    n = NCB * 8
    iot = lax.broadcasted_iota(jnp.int32, (n, B), 0)
    big = jnp.int32(2**30)
    for t in range(TOPS):
        m = jnp.max(vals, axis=0, keepdims=True)
        pos = jnp.min(jnp.where(vals == m, iot, big), axis=0, keepdims=True)
        hit = iot == pos
        o_ref[t:t + 1, :] = jnp.sum(jnp.where(hit, ids, 0), axis=0,
                                    keepdims=True)
        vals = jnp.where(hit, NEG, vals)
    o_ref[TOPS:8, :] = jnp.zeros((8 - TOPS, B), jnp.int32)


def _stage_top6(vals6, ids6):
    return pl.pallas_call(
        _merge_body,
        in_specs=[
            pl.BlockSpec((NCB * 8, B), lambda: (0, 0)),
            pl.BlockSpec((NCB * 8, B), lambda: (0, 0)),
        ],
        out_specs=pl.BlockSpec((8, B), lambda: (0, 0)),
        out_shape=jax.ShapeDtypeStruct((8, B), jnp.int32),
    )(vals6, ids6)


@functools.cache
def _make_sc_gather():
    @functools.partial(
        pl.kernel,
        mesh=plsc.VectorSubcoreMesh(core_axis_name="c", subcore_axis_name="s"),
        out_type=jax.ShapeDtypeStruct((B, REQ, STR, 128), jnp.float32),
        scratch_types=[
            pltpu.VMEM((SPW, 16), jnp.int32),
            pltpu.VMEM((HALFS, REQ, STR, 128), jnp.float32),
            pltpu.SemaphoreType.DMA,
        ],
    )
    def gather_k(xt_hbm, rows_hbm, out_hbm, rows_v, tiles_v, sem):
        wid = lax.axis_index("s") * 2 + lax.axis_index("c")
        lane0 = pl.multiple_of((wid // 4) * 128, 128)
        pltpu.sync_copy(rows_hbm.at[wid], rows_v)
        for h in range(2):
            copies = []
            for sl in range(HALFS):
                s = h * HALFS + sl
                rvec = rows_v[s]
                for j in range(REQ):
                    r0 = pl.multiple_of(rvec[j], STR)
                    copies.append(pltpu.async_copy(
                        xt_hbm.at[pl.ds(r0, STR), pl.ds(lane0, 128)],
                        tiles_v.at[sl, j],
                        sem,
                    ))
            for cp in copies:
                cp.wait()
            pltpu.sync_copy(
                tiles_v,
                out_hbm.at[pl.ds(wid * SPW + h * HALFS, HALFS)],
            )

    return gather_k


def _sc_gather(xt, rows):
    return _make_sc_gather()(xt, rows)


def _final_body(g_ref, ids_ref, y_ref, o_ref):
    c = pl.program_id(0)
    y = y_ref[...]
    g = g_ref[...]
    sl = lax.broadcasted_iota(jnp.int32, (FB, 1, 1, 1), 0)
    lane = lax.broadcasted_iota(jnp.int32, (FB, REQ, STR, 128), 3)
    gd = jnp.max(jnp.where(lane == sl, g, NEG), axis=3)

    jdim = lax.broadcasted_iota(jnp.int32, (FB, REQ, STR), 1)
    row8 = lax.broadcasted_iota(jnp.int32, (FB, REQ, STR), 2)
    sj = jnp.concatenate(
        [jnp.broadcast_to(ids_ref[:, t:t + 1][:, :, None], (FB, 1, STR))
         for t in range(TOPS)]
        + [jnp.zeros((FB, 1, STR), jnp.int32)], axis=1
    )
    col = sj * STR + row8
    y3 = y[:, :, None]
    valid = (jdim < TOPS) & (col != y3)
    cand = jnp.where(valid, gd, NEG)

    pid = jdim * STR + row8
    big = jnp.int32(2**30)
    for _ in range(K - 1):
        m = jnp.max(jnp.max(cand, axis=2), axis=1)[:, None, None]
        idx = jnp.min(jnp.min(
            jnp.where(cand == m, pid, big), axis=2), axis=1)[:, None, None]
        cand = jnp.where(pid == idx, NEG, cand)
    tk = jnp.max(jnp.max(cand, axis=2), axis=1, keepdims=True)

    x2win = gd[:, TOPS, :]
    i8 = lax.broadcasted_iota(jnp.int32, (FB, STR), 1)
    x2 = jnp.sum(
        jnp.where(i8 == jnp.bitwise_and(y, STR - 1), x2win, 0.0),
        axis=1, keepdims=True,
    )

    loss = jnp.maximum(ALPHA + (tk - x2) * (1.0 / K), 0.0)
    part = jnp.sum(loss, keepdims=True)[:1, :1] * (1.0 / B)

    @pl.when(c == 0)
    def _init():
        o_ref[...] = part

    @pl.when(c > 0)
    def _acc():
        o_ref[...] = o_ref[...] + part


def _stage_final(g, ids, y2):
    nblk = B // FB
    return pl.pallas_call(
        _final_body,
        grid=(nblk,),
        in_specs=[
            pl.BlockSpec((FB, REQ, STR, 128), lambda c: (c, 0, 0, 0)),
            pl.BlockSpec((FB, 8), lambda c: (c, 0)),
            pl.BlockSpec((FB, 1), lambda c: (c, 0)),
        ],
        out_specs=pl.BlockSpec((1, 1), lambda c: (0, 0)),
        out_shape=jax.ShapeDtypeStruct((1, 1), jnp.float32),
    )(g, ids, y2)


def kernel(x, y):
    xt = x.T
    vals6, ids6 = _stage_stripemax(xt)
    idsT = _stage_top6(vals6, ids6)
    ids = idsT.T

    y32 = y.astype(jnp.int32)
    rows = jnp.concatenate(
        [ids[:, :TOPS] * STR, jnp.bitwise_and(y32, -STR)[:, None],
         jnp.zeros((B, 9), jnp.int32)], axis=1
    )
    rows = rows.reshape(NW, SPW, 16)

    g = _sc_gather(xt, rows)

    out = _stage_final(g, ids, y32[:, None])
    return out[0, 0]


# speedup vs baseline: 1.0846x; 1.0846x over previous
import functools

import jax
import jax.numpy as jnp
from jax import lax
from jax.experimental import pallas as pl
from jax.experimental.pallas import tpu as pltpu
from jax.experimental.pallas import tpu_sc as plsc

B = 1024
C = 100000
K = 5
ALPHA = 1.0
TOPS = K + 1
REQ = TOPS + 1

STR = 8
NSTR = C // STR
CBLK = 4096
NCB = -(-C // CBLK)
SPB = CBLK // STR

NW = 32
SPW = B // NW
NEG = float("-inf")


NSTR_PAD = NCB * SPB


def _stripemax_body(x_ref, o_ref):
    j = pl.program_id(0)
    xb = x_ref[...]
    x3 = xb.reshape(SPB, STR, B)

    @pl.when(j < NCB - 1)
    def _full():
        o_ref[...] = jnp.max(x3, axis=1)

    @pl.when(j == NCB - 1)
    def _tail():
        r = (lax.broadcasted_iota(jnp.int32, (SPB, STR, B), 0) * STR
             + lax.broadcasted_iota(jnp.int32, (SPB, STR, B), 1))
        xm = jnp.where(j * CBLK + r < C, x3, NEG)
        o_ref[...] = jnp.max(xm, axis=1)


def _stage_stripemax(xt):
    return pl.pallas_call(
        _stripemax_body,
        grid=(NCB,),
        in_specs=[pl.BlockSpec((CBLK, B), lambda j: (j, 0))],
        out_specs=pl.BlockSpec((SPB, B), lambda j: (j, 0)),
        out_shape=jax.ShapeDtypeStruct((NSTR_PAD, B), jnp.float32),
    )(xt)


TB = 256


def _top6_body(s_ref, o_ref):
    vals = s_ref[...]
    iot = lax.broadcasted_iota(jnp.int32, (NSTR_PAD, TB), 0)
    big = jnp.int32(2**30)
    for t in range(TOPS):
        m = jnp.max(vals, axis=0, keepdims=True)
        idx = jnp.min(jnp.where(vals == m, iot, big), axis=0, keepdims=True)
        o_ref[t:t + 1, :] = idx
        vals = jnp.where(iot == idx, NEG, vals)
    o_ref[TOPS:8, :] = jnp.zeros((8 - TOPS, TB), jnp.int32)


def _stage_top6(stripemax):
    return pl.pallas_call(
        _top6_body,
        grid=(B // TB,),
        in_specs=[pl.BlockSpec((NSTR_PAD, TB), lambda c: (0, c))],
        out_specs=pl.BlockSpec((8, TB), lambda c: (0, c)),
        out_shape=jax.ShapeDtypeStruct((8, B), jnp.int32),
    )(stripemax)


HALFS = SPW // 2
FB = 128


@functools.cache
def _make_sc_gather():
    @functools.partial(
        pl.kernel,
        mesh=plsc.VectorSubcoreMesh(core_axis_name="c", subcore_axis_name="s"),
        out_type=jax.ShapeDtypeStruct((B, REQ, STR, 128), jnp.float32),
        scratch_types=[
            pltpu.VMEM((SPW, 16), jnp.int32),
            pltpu.VMEM((HALFS, REQ, STR, 128), jnp.float32),
            pltpu.SemaphoreType.DMA,
        ],
    )
    def gather_k(xt_hbm, rows_hbm, out_hbm, rows_v, tiles_v, sem):
        wid = lax.axis_index("s") * 2 + lax.axis_index("c")
        lane0 = pl.multiple_of((wid // 4) * 128, 128)
        pltpu.sync_copy(rows_hbm.at[wid], rows_v)
        for h in range(2):
            copies = []
            for sl in range(HALFS):
                s = h * HALFS + sl
                rvec = rows_v[s]
                for j in range(REQ):
                    r0 = pl.multiple_of(rvec[j], STR)
                    copies.append(pltpu.async_copy(
                        xt_hbm.at[pl.ds(r0, STR), pl.ds(lane0, 128)],
                        tiles_v.at[sl, j],
                        sem,
                    ))
            for cp in copies:
                cp.wait()
            pltpu.sync_copy(
                tiles_v,
                out_hbm.at[pl.ds(wid * SPW + h * HALFS, HALFS)],
            )

    return gather_k


def _sc_gather(xt, rows):
    return _make_sc_gather()(xt, rows)


def _final_body(g_ref, ids_ref, y_ref, o_ref):
    c = pl.program_id(0)
    y = y_ref[...]
    g = g_ref[...]
    sl = lax.broadcasted_iota(jnp.int32, (FB, 1, 1, 1), 0)
    lane = lax.broadcasted_iota(jnp.int32, (FB, REQ, STR, 128), 3)
    gd = jnp.max(jnp.where(lane == sl, g, NEG), axis=3)

    jdim = lax.broadcasted_iota(jnp.int32, (FB, REQ, STR), 1)
    row8 = lax.broadcasted_iota(jnp.int32, (FB, REQ, STR), 2)
    sj = jnp.concatenate(
        [jnp.broadcast_to(ids_ref[:, t:t + 1][:, :, None], (FB, 1, STR))
         for t in range(TOPS)]
        + [jnp.zeros((FB, 1, STR), jnp.int32)], axis=1
    )
    col = sj * STR + row8
    y3 = y[:, :, None]
    valid = (jdim < TOPS) & (col != y3)
    cand = jnp.where(valid, gd, NEG)

    pid = jdim * STR + row8
    big = jnp.int32(2**30)
    for _ in range(K - 1):
        m = jnp.max(jnp.max(cand, axis=2), axis=1)[:, None, None]
        idx = jnp.min(jnp.min(
            jnp.where(cand == m, pid, big), axis=2), axis=1)[:, None, None]
        cand = jnp.where(pid == idx, NEG, cand)
    tk = jnp.max(jnp.max(cand, axis=2), axis=1, keepdims=True)

    x2win = gd[:, TOPS, :]
    i8 = lax.broadcasted_iota(jnp.int32, (FB, STR), 1)
    x2 = jnp.sum(
        jnp.where(i8 == jnp.bitwise_and(y, STR - 1), x2win, 0.0),
        axis=1, keepdims=True,
    )

    loss = jnp.maximum(ALPHA + (tk - x2) * (1.0 / K), 0.0)
    part = jnp.sum(loss, keepdims=True)[:1, :1] * (1.0 / B)

    @pl.when(c == 0)
    def _init():
        o_ref[...] = part

    @pl.when(c > 0)
    def _acc():
        o_ref[...] = o_ref[...] + part


def _stage_final(g, ids, y2):
    nblk = B // FB
    return pl.pallas_call(
        _final_body,
        grid=(nblk,),
        in_specs=[
            pl.BlockSpec((FB, REQ, STR, 128), lambda c: (c, 0, 0, 0)),
            pl.BlockSpec((FB, 8), lambda c: (c, 0)),
            pl.BlockSpec((FB, 1), lambda c: (c, 0)),
        ],
        out_specs=pl.BlockSpec((1, 1), lambda c: (0, 0)),
        out_shape=jax.ShapeDtypeStruct((1, 1), jnp.float32),
    )(g, ids, y2)


def kernel(x, y):
    xt = x.T
    stripemax = _stage_stripemax(xt)
    idsT = _stage_top6(stripemax)
    ids = idsT.T

    y32 = y.astype(jnp.int32)
    rows = jnp.concatenate(
        [ids[:, :TOPS] * STR, jnp.bitwise_and(y32, -STR)[:, None],
         jnp.zeros((B, 9), jnp.int32)], axis=1
    )
    rows = rows.reshape(NW, SPW, 16)

    g = _sc_gather(xt, rows)

    out = _stage_final(g, ids, y32[:, None])
    return out[0, 0]
